# Initial kernel scaffold; baseline (speedup 1.0000x reference)
import jax
import jax.numpy as jnp
from jax.experimental import pallas as pl


def kernel(x, attr, W_gcn, b_gcn, Wl0, bl0, Wr0, Wl1, bl1, Wr1, edge_index_0, e_id_0, edge_index_1, e_id_1):
    raise NotImplementedError("write your pallas kernel here")


# speedup vs baseline: 11.2944x; 11.2944x over previous
import functools

import jax
import jax.numpy as jnp
from jax import lax
from jax.experimental import pallas as pl
from jax.experimental.pallas import tpu as pltpu
from jax.experimental.pallas import tpu_sc as plsc

N1 = 1000
E1 = 100000
E0 = 320000
NC, NS, L = 2, 16, 16
EPT = 6400
NCHUNK = EPT // 128
NPH = 2
HCOL = N1 // NPH
DUMP = HCOL * N1
ACC = 500224
SLICE = ACC // NS


def _sc_body(row_hbm, col_hbm, eid_hbm, attr_hbm, acc_out,
             row_v, col_v, eid_v, flat_v, val_v, stage_v, acc_sh, sem):
    c = lax.axis_index("c")
    s = lax.axis_index("s")

    pltpu.sync_copy(row_hbm.at[s], row_v)
    pltpu.sync_copy(col_hbm.at[s], col_v)

    def vbody(j, carry):
        def kbody(k, carry2):
            val_v[j, pl.ds(k * L, L)] = jnp.full((L,), 1.0, jnp.float32)
            return carry2
        return lax.fori_loop(0, 128 // L, kbody, carry)
    lax.fori_loop(0, NCHUNK, vbody, 0)

    @pl.when(c == 0)
    def _gather_w():
        pltpu.sync_copy(eid_hbm.at[s], eid_v)
        for j in range(NCHUNK):
            pltpu.async_copy(attr_hbm.at[eid_v.at[j]], val_v.at[j], sem).wait()

    for p in range(NPH):
        def zfill(i, carry):
            stage_v[pl.ds(i * L, L)] = jnp.zeros((L,), jnp.float32)
            return carry
        lax.fori_loop(0, SLICE // L, zfill, 0)
        pltpu.sync_copy(stage_v, acc_sh.at[pl.ds(s * SLICE, SLICE)])

        def fbody(j, carry):
            def kbody(k, carry2):
                off = k * L
                r = row_v[j, pl.ds(off, L)]
                cc = col_v[j, pl.ds(off, L)] - (p * HCOL)
                flat = cc * N1 + r
                ok = (cc >= 0) & (cc < HCOL)
                flat_v[j, pl.ds(off, L)] = jnp.where(
                    ok, flat, jnp.full((L,), DUMP, jnp.int32))
                return carry2
            return lax.fori_loop(0, 128 // L, kbody, carry)
        lax.fori_loop(0, NCHUNK, fbody, 0)
        plsc.subcore_barrier()

        for j in range(NCHUNK):
            pltpu.sync_copy(val_v.at[j], acc_sh.at[flat_v.at[j]], add=True)
        plsc.subcore_barrier()

        pltpu.sync_copy(acc_sh.at[pl.ds(s * SLICE, SLICE)], stage_v)
        pltpu.sync_copy(
            stage_v,
            acc_out.at[pl.ds((c * NPH + p) * ACC + s * SLICE, SLICE)])


def _make_sc_build():
    return functools.partial(
        pl.kernel,
        out_type=jax.ShapeDtypeStruct((NC * NPH * ACC,), jnp.float32),
        mesh=plsc.VectorSubcoreMesh(core_axis_name="c", subcore_axis_name="s",
                                    num_cores=NC, num_subcores=NS),
        scratch_types=[
            pltpu.VMEM((NCHUNK, 128), jnp.int32),
            pltpu.VMEM((NCHUNK, 128), jnp.int32),
            pltpu.VMEM((NCHUNK, 128), jnp.int32),
            pltpu.VMEM((NCHUNK, 128), jnp.int32),
            pltpu.VMEM((NCHUNK, 128), jnp.float32),
            pltpu.VMEM((SLICE,), jnp.float32),
            pltpu.VMEM_SHARED((ACC,), jnp.float32),
            pltpu.SemaphoreType.DMA,
        ],
    )(_sc_body)


def _tc_body(wsum_ref, adj_ref, x_ref, wgcn_ref, bgcn_ref,
             wl_ref, bl_ref, wr_ref, out_ref):
    wt = wsum_ref[...]
    at = adj_ref[...]
    deg = 1.0 + jnp.sum(wt, axis=1, keepdims=True)
    dinv = jnp.where(deg > 0, lax.rsqrt(deg), 0.0)
    h1 = jnp.dot(x_ref[...], wgcn_ref[...], preferred_element_type=jnp.float32)
    u = jnp.dot(wt, dinv * h1, preferred_element_type=jnp.float32)
    hg = dinv * u + h1 / deg + bgcn_ref[...]
    cnt = jnp.sum(at, axis=1, keepdims=True)
    sagg = jnp.dot(at, hg, preferred_element_type=jnp.float32)
    mean = sagg / jnp.maximum(cnt, 1.0)
    o = (jnp.dot(mean, wl_ref[...], preferred_element_type=jnp.float32)
         + bl_ref[...]
         + jnp.dot(hg, wr_ref[...], preferred_element_type=jnp.float32))
    nrm = jnp.sqrt(jnp.sum(o * o, axis=1, keepdims=True))
    out_ref[...] = o / jnp.maximum(nrm, 1e-12)


def _tc_dense(wsum, adj, x1k, W_gcn, b_gcn, Wl1, bl1, Wr1):
    return pl.pallas_call(
        _tc_body,
        out_shape=jax.ShapeDtypeStruct((N1, 128), jnp.float32),
    )(wsum, adj, x1k, W_gcn, b_gcn, Wl1, bl1, Wr1)


def _half(acc, k):
    return acc[k * ACC:k * ACC + HCOL * N1].reshape(HCOL, N1)


def kernel(x, attr, W_gcn, b_gcn, Wl0, bl0, Wr0, Wl1, bl1, Wr1,
           edge_index_0, e_id_0, edge_index_1, e_id_1):
    del Wl0, bl0, Wr0, edge_index_0, e_id_0
    ep = NS * EPT
    pad = ep - E1
    row = jnp.concatenate([edge_index_1[0], jnp.zeros((pad,), jnp.int32)])
    col = jnp.concatenate([edge_index_1[1], jnp.full((pad,), N1, jnp.int32)])
    eid = jnp.concatenate([e_id_1, jnp.full((pad,), E0, jnp.int32)])
    attr_ext = jnp.concatenate([attr.reshape(-1), jnp.zeros((8,), jnp.float32)])
    row = row.reshape(NS, NCHUNK, 128)
    col = col.reshape(NS, NCHUNK, 128)
    eid = eid.reshape(NS, NCHUNK, 128)

    acc = _make_sc_build()(row, col, eid, attr_ext)
    wsum = jnp.concatenate([_half(acc, 0), _half(acc, 1)], axis=0)
    adj = jnp.concatenate([_half(acc, 2), _half(acc, 3)], axis=0)

    return _tc_dense(wsum, adj, x[:N1],
                     W_gcn, b_gcn.reshape(1, 128), Wl1, bl1.reshape(1, 128), Wr1)


# speedup vs baseline: 13.1479x; 1.1641x over previous
import functools

import jax
import jax.numpy as jnp
from jax import lax
from jax.experimental import pallas as pl
from jax.experimental.pallas import tpu as pltpu
from jax.experimental.pallas import tpu_sc as plsc

N1 = 1000
E1 = 100000
E0 = 320000
NC, NS, L = 2, 16, 16
EPT = 6400
NCHUNK = EPT // 128
NPH = 2
HCOL = N1 // NPH
DUMP = HCOL * N1
ACC = 500224
SLICE = ACC // NS


def _sc_body(row_hbm, col_hbm, eid_hbm, attr_hbm, zeros_hbm, ones_hbm,
             acc_out,
             row_v, col_v, eid_v, flat_v, val_v, zbuf_v, stage_v, acc_sh,
             sem, gsem):
    c = lax.axis_index("c")
    s = lax.axis_index("s")

    ld = [pltpu.make_async_copy(row_hbm.at[s], row_v, sem),
          pltpu.make_async_copy(col_hbm.at[s], col_v, sem),
          pltpu.make_async_copy(zeros_hbm, zbuf_v, sem),
          pltpu.make_async_copy(ones_hbm, val_v, sem)]
    for d in ld:
        d.start()
    for d in ld:
        d.wait()

    @pl.when(c == 0)
    def _gather_w():
        pltpu.sync_copy(eid_hbm.at[s], eid_v)
        gs = [pltpu.make_async_copy(attr_hbm.at[eid_v.at[j]], val_v.at[j],
                                    gsem)
              for j in range(NCHUNK)]
        for d in gs:
            d.start()
        for d in gs:
            d.wait()

    outb = None
    for p in range(NPH):
        pltpu.sync_copy(zbuf_v, acc_sh.at[pl.ds(s * SLICE, SLICE)])

        def fbody(j, carry):
            def kbody(k, carry2):
                off = k * L
                r = row_v[j, pl.ds(off, L)]
                cc = col_v[j, pl.ds(off, L)] - (p * HCOL)
                flat = cc * N1 + r
                ok = (cc >= 0) & (cc < HCOL)
                flat_v[j, pl.ds(off, L)] = jnp.where(
                    ok, flat, jnp.full((L,), DUMP, jnp.int32))
                return carry2
            return lax.fori_loop(0, 128 // L, kbody, carry)
        lax.fori_loop(0, NCHUNK, fbody, 0)
        plsc.subcore_barrier()

        sc = [pltpu.make_async_copy(val_v.at[j], acc_sh.at[flat_v.at[j]],
                                    sem)
              for j in range(NCHUNK)]
        for d in sc:
            d.start(add=True)
        for d in sc:
            d.wait()
        plsc.subcore_barrier()

        if outb is not None:
            outb.wait()
        pltpu.sync_copy(acc_sh.at[pl.ds(s * SLICE, SLICE)], stage_v)
        outb = pltpu.make_async_copy(
            stage_v,
            acc_out.at[pl.ds((c * NPH + p) * ACC + s * SLICE, SLICE)],
            gsem)
        outb.start()
    outb.wait()


def _make_sc_build():
    return functools.partial(
        pl.kernel,
        out_type=jax.ShapeDtypeStruct((NC * NPH * ACC,), jnp.float32),
        mesh=plsc.VectorSubcoreMesh(core_axis_name="c", subcore_axis_name="s",
                                    num_cores=NC, num_subcores=NS),
        scratch_types=[
            pltpu.VMEM((NCHUNK, 128), jnp.int32),
            pltpu.VMEM((NCHUNK, 128), jnp.int32),
            pltpu.VMEM((NCHUNK, 128), jnp.int32),
            pltpu.VMEM((NCHUNK, 128), jnp.int32),
            pltpu.VMEM((NCHUNK, 128), jnp.float32),
            pltpu.VMEM((SLICE,), jnp.float32),
            pltpu.VMEM((SLICE,), jnp.float32),
            pltpu.VMEM_SHARED((ACC,), jnp.float32),
            pltpu.SemaphoreType.DMA,
            pltpu.SemaphoreType.DMA,
        ],
    )(_sc_body)


def _tc_body(wsum_ref, adj_ref, x_ref, wgcn_ref, bgcn_ref,
             wl_ref, bl_ref, wr_ref, out_ref):
    wt = wsum_ref[...]
    at = adj_ref[...]
    deg = 1.0 + jnp.sum(wt, axis=1, keepdims=True)
    dinv = jnp.where(deg > 0, lax.rsqrt(deg), 0.0)
    h1 = jnp.dot(x_ref[...], wgcn_ref[...], preferred_element_type=jnp.float32)
    u = jnp.dot(wt, dinv * h1, preferred_element_type=jnp.float32)
    hg = dinv * u + h1 / deg + bgcn_ref[...]
    cnt = jnp.sum(at, axis=1, keepdims=True)
    sagg = jnp.dot(at, hg, preferred_element_type=jnp.float32)
    mean = sagg / jnp.maximum(cnt, 1.0)
    o = (jnp.dot(mean, wl_ref[...], preferred_element_type=jnp.float32)
         + bl_ref[...]
         + jnp.dot(hg, wr_ref[...], preferred_element_type=jnp.float32))
    nrm = jnp.sqrt(jnp.sum(o * o, axis=1, keepdims=True))
    out_ref[...] = o / jnp.maximum(nrm, 1e-12)


def _tc_dense(wsum, adj, x1k, W_gcn, b_gcn, Wl1, bl1, Wr1):
    return pl.pallas_call(
        _tc_body,
        out_shape=jax.ShapeDtypeStruct((N1, 128), jnp.float32),
    )(wsum, adj, x1k, W_gcn, b_gcn, Wl1, bl1, Wr1)


def _half(acc, k):
    return acc[k * ACC:k * ACC + HCOL * N1].reshape(HCOL, N1)


def kernel(x, attr, W_gcn, b_gcn, Wl0, bl0, Wr0, Wl1, bl1, Wr1,
           edge_index_0, e_id_0, edge_index_1, e_id_1):
    del Wl0, bl0, Wr0, edge_index_0, e_id_0
    ep = NS * EPT
    pad = ep - E1
    row = jnp.concatenate([edge_index_1[0], jnp.zeros((pad,), jnp.int32)])
    col = jnp.concatenate([edge_index_1[1], jnp.full((pad,), N1, jnp.int32)])
    eid = jnp.concatenate([e_id_1, jnp.full((pad,), E0, jnp.int32)])
    attr_ext = jnp.concatenate([attr.reshape(-1), jnp.zeros((8,), jnp.float32)])
    row = row.reshape(NS, NCHUNK, 128)
    col = col.reshape(NS, NCHUNK, 128)
    eid = eid.reshape(NS, NCHUNK, 128)

    zeros_in = jnp.zeros((SLICE,), jnp.float32)
    ones_in = jnp.ones((NCHUNK, 128), jnp.float32)
    acc = _make_sc_build()(row, col, eid, attr_ext, zeros_in, ones_in)
    wsum = jnp.concatenate([_half(acc, 0), _half(acc, 1)], axis=0)
    adj = jnp.concatenate([_half(acc, 2), _half(acc, 3)], axis=0)

    return _tc_dense(wsum, adj, x[:N1],
                     W_gcn, b_gcn.reshape(1, 128), Wl1, bl1.reshape(1, 128), Wr1)


# speedup vs baseline: 13.6076x; 1.0350x over previous
import functools

import jax
import jax.numpy as jnp
from jax import lax
from jax.experimental import pallas as pl
from jax.experimental.pallas import tpu as pltpu
from jax.experimental.pallas import tpu_sc as plsc

N1 = 1000
E1 = 100000
E0 = 320000
NC, NS, L = 2, 16, 16
EPT = 6400
NCHUNK = EPT // 128
NPH = 2
HCOL = N1 // NPH
DUMP = HCOL * N1
ACC = 500224
SLICE = ACC // NS
CORE_OUT = N1 * N1 + (ACC - HCOL * N1)


def _sc_body(row_hbm, col_hbm, eid_hbm, attr_hbm, zeros_hbm, ones_hbm,
             acc_out,
             row_v, col_v, eid_v, flat_v, val_v, zbuf_v, stage_v, acc_sh,
             sem, gsem):
    c = lax.axis_index("c")
    s = lax.axis_index("s")

    ld = [pltpu.make_async_copy(row_hbm.at[s], row_v, sem),
          pltpu.make_async_copy(col_hbm.at[s], col_v, sem),
          pltpu.make_async_copy(zeros_hbm, zbuf_v, sem),
          pltpu.make_async_copy(ones_hbm, val_v, sem)]
    for d in ld:
        d.start()
    for d in ld:
        d.wait()

    @pl.when(c == 0)
    def _gather_w():
        pltpu.sync_copy(eid_hbm.at[s], eid_v)
        gs = [pltpu.make_async_copy(attr_hbm.at[eid_v.at[j]], val_v.at[j],
                                    gsem)
              for j in range(NCHUNK)]
        for d in gs:
            d.start()
        for d in gs:
            d.wait()

    outb = None
    for p in range(NPH):
        pltpu.sync_copy(zbuf_v, acc_sh.at[pl.ds(s * SLICE, SLICE)])

        def fbody(j, carry):
            def kbody(k, carry2):
                off = k * L
                r = row_v[j, pl.ds(off, L)]
                cc = col_v[j, pl.ds(off, L)] - (p * HCOL)
                flat = cc * N1 + r
                ok = (cc >= 0) & (cc < HCOL)
                flat_v[j, pl.ds(off, L)] = jnp.where(
                    ok, flat, jnp.full((L,), DUMP, jnp.int32))
                return carry2
            return lax.fori_loop(0, 128 // L, kbody, carry)
        lax.fori_loop(0, NCHUNK, fbody, 0)
        plsc.subcore_barrier()

        sc = [pltpu.make_async_copy(val_v.at[j], acc_sh.at[flat_v.at[j]],
                                    sem)
              for j in range(NCHUNK)]
        for d in sc:
            d.start(add=True)
        for d in sc:
            d.wait()
        if outb is not None:
            outb.wait()
        plsc.subcore_barrier()

        pltpu.sync_copy(acc_sh.at[pl.ds(s * SLICE, SLICE)], stage_v)
        outb = pltpu.make_async_copy(
            stage_v,
            acc_out.at[pl.ds(c * CORE_OUT + p * HCOL * N1 + s * SLICE,
                             SLICE)],
            gsem)
        outb.start()
    outb.wait()


def _make_sc_build():
    return functools.partial(
        pl.kernel,
        out_type=jax.ShapeDtypeStruct((NC * CORE_OUT,), jnp.float32),
        mesh=plsc.VectorSubcoreMesh(core_axis_name="c", subcore_axis_name="s",
                                    num_cores=NC, num_subcores=NS),
        scratch_types=[
            pltpu.VMEM((NCHUNK, 128), jnp.int32),
            pltpu.VMEM((NCHUNK, 128), jnp.int32),
            pltpu.VMEM((NCHUNK, 128), jnp.int32),
            pltpu.VMEM((NCHUNK, 128), jnp.int32),
            pltpu.VMEM((NCHUNK, 128), jnp.float32),
            pltpu.VMEM((SLICE,), jnp.float32),
            pltpu.VMEM((SLICE,), jnp.float32),
            pltpu.VMEM_SHARED((ACC,), jnp.float32),
            pltpu.SemaphoreType.DMA,
            pltpu.SemaphoreType.DMA,
        ],
    )(_sc_body)


def _tc_body(wsum_ref, adj_ref, x_ref, wgcn_ref, bgcn_ref,
             wl_ref, bl_ref, wr_ref, out_ref):
    wt = wsum_ref[...]
    at = adj_ref[...]
    deg = 1.0 + jnp.sum(wt, axis=1, keepdims=True)
    dinv = jnp.where(deg > 0, lax.rsqrt(deg), 0.0)
    h1 = jnp.dot(x_ref[...], wgcn_ref[...], preferred_element_type=jnp.float32)
    u = jnp.dot(wt, dinv * h1, preferred_element_type=jnp.float32)
    hg = dinv * u + h1 / deg + bgcn_ref[...]
    cnt = jnp.sum(at, axis=1, keepdims=True)
    sagg = jnp.dot(at, hg, preferred_element_type=jnp.float32)
    mean = sagg / jnp.maximum(cnt, 1.0)
    o = (jnp.dot(mean, wl_ref[...], preferred_element_type=jnp.float32)
         + bl_ref[...]
         + jnp.dot(hg, wr_ref[...], preferred_element_type=jnp.float32))
    nrm = jnp.sqrt(jnp.sum(o * o, axis=1, keepdims=True))
    out_ref[...] = o / jnp.maximum(nrm, 1e-12)


def _tc_dense(wsum, adj, x1k, W_gcn, b_gcn, Wl1, bl1, Wr1):
    return pl.pallas_call(
        _tc_body,
        out_shape=jax.ShapeDtypeStruct((N1, 128), jnp.float32),
    )(wsum, adj, x1k, W_gcn, b_gcn, Wl1, bl1, Wr1)


def kernel(x, attr, W_gcn, b_gcn, Wl0, bl0, Wr0, Wl1, bl1, Wr1,
           edge_index_0, e_id_0, edge_index_1, e_id_1):
    del Wl0, bl0, Wr0, edge_index_0, e_id_0
    ep = NS * EPT
    pad = ep - E1
    row = jnp.concatenate([edge_index_1[0], jnp.zeros((pad,), jnp.int32)])
    col = jnp.concatenate([edge_index_1[1], jnp.full((pad,), N1, jnp.int32)])
    eid = jnp.concatenate([e_id_1, jnp.zeros((pad,), jnp.int32)])
    row = row.reshape(NS, NCHUNK, 128)
    col = col.reshape(NS, NCHUNK, 128)
    eid = eid.reshape(NS, NCHUNK, 128)

    zeros_in = jnp.zeros((SLICE,), jnp.float32)
    ones_in = jnp.ones((NCHUNK, 128), jnp.float32)
    acc = _make_sc_build()(row, col, eid, attr.reshape(-1), zeros_in, ones_in)
    wsum = acc[:N1 * N1].reshape(N1, N1)
    adj = acc[CORE_OUT:CORE_OUT + N1 * N1].reshape(N1, N1)

    return _tc_dense(wsum, adj, x[:N1],
                     W_gcn, b_gcn.reshape(1, 128), Wl1, bl1.reshape(1, 128), Wr1)


# speedup vs baseline: 28.1688x; 2.0701x over previous
import functools

import jax
import jax.numpy as jnp
from jax import lax
from jax.experimental import pallas as pl
from jax.experimental.pallas import tpu as pltpu
from jax.experimental.pallas import tpu_sc as plsc

N1 = 1000
E1 = 100000
E0 = 320000
NC, NS, L = 2, 16, 16
EPT = 6400
NCHUNK = EPT // 128
NPH = 2
HCOL = N1 // NPH
DUMP = HCOL * N1
ACC = 500224
SLICE = ACC // NS
CORE_OUT = N1 * N1 + (ACC - HCOL * N1)


def _sc_body(row_hbm, col_hbm, eid_hbm, attr_hbm, zeros_hbm, ones_hbm,
             acc_out,
             row_v, col_v, eid_v, flat_v, val_v, zbuf_v, stage_v, acc_sh,
             sem, gsem):
    c = lax.axis_index("c")
    s = lax.axis_index("s")

    dump_lane = lax.iota(jnp.int32, L)
    ld = [pltpu.make_async_copy(row_hbm.at[s], row_v, sem),
          pltpu.make_async_copy(col_hbm.at[s], col_v, sem),
          pltpu.make_async_copy(zeros_hbm, zbuf_v, sem),
          pltpu.make_async_copy(ones_hbm, val_v, sem)]
    for d in ld:
        d.start()
    for d in ld:
        d.wait()

    @pl.when(c == 0)
    def _gather_w():
      with jax.named_scope("sc_gather_w"):
        pltpu.sync_copy(eid_hbm.at[s], eid_v)
        gs = [pltpu.make_async_copy(attr_hbm.at[eid_v.at[j]], val_v.at[j],
                                    gsem)
              for j in range(NCHUNK)]
        for d in gs:
            d.start()
        for d in gs:
            d.wait()

    outb = None
    for p in range(NPH):
      with jax.named_scope(f"sc_zero_{p}"):
        pltpu.sync_copy(zbuf_v, acc_sh.at[pl.ds(s * SLICE, SLICE)])

      with jax.named_scope(f"sc_flat_{p}"):
        def fbody(j, carry):
            def kbody(k, carry2):
                off = k * L
                r = row_v[j, pl.ds(off, L)]
                cc = col_v[j, pl.ds(off, L)] - (p * HCOL)
                flat = cc * N1 + r
                ok = (cc >= 0) & (cc < HCOL)
                flat_v[j, pl.ds(off, L)] = jnp.where(
                    ok, flat, DUMP + off + dump_lane)
                return carry2
            return lax.fori_loop(0, 128 // L, kbody, carry)
        lax.fori_loop(0, NCHUNK, fbody, 0)
        plsc.subcore_barrier()

      with jax.named_scope(f"sc_scatter_{p}"):
        sc = [pltpu.make_async_copy(val_v.at[j], acc_sh.at[flat_v.at[j]],
                                    sem)
              for j in range(NCHUNK)]
        for d in sc:
            d.start(add=True)
        for d in sc:
            d.wait()
        if outb is not None:
            outb.wait()
        plsc.subcore_barrier()

      with jax.named_scope(f"sc_out_{p}"):
        pltpu.sync_copy(acc_sh.at[pl.ds(s * SLICE, SLICE)], stage_v)
        outb = pltpu.make_async_copy(
            stage_v,
            acc_out.at[pl.ds(c * CORE_OUT + p * HCOL * N1 + s * SLICE,
                             SLICE)],
            gsem)
        outb.start()
    outb.wait()


def _make_sc_build():
    return functools.partial(
        pl.kernel,
        out_type=jax.ShapeDtypeStruct((NC * CORE_OUT,), jnp.float32),
        mesh=plsc.VectorSubcoreMesh(core_axis_name="c", subcore_axis_name="s",
                                    num_cores=NC, num_subcores=NS),
        scratch_types=[
            pltpu.VMEM((NCHUNK, 128), jnp.int32),
            pltpu.VMEM((NCHUNK, 128), jnp.int32),
            pltpu.VMEM((NCHUNK, 128), jnp.int32),
            pltpu.VMEM((NCHUNK, 128), jnp.int32),
            pltpu.VMEM((NCHUNK, 128), jnp.float32),
            pltpu.VMEM((SLICE,), jnp.float32),
            pltpu.VMEM((SLICE,), jnp.float32),
            pltpu.VMEM_SHARED((ACC,), jnp.float32),
            pltpu.SemaphoreType.DMA,
            pltpu.SemaphoreType.DMA,
        ],
    )(_sc_body)


def _tc_body(wsum_ref, adj_ref, x_ref, wgcn_ref, bgcn_ref,
             wl_ref, bl_ref, wr_ref, out_ref):
    wt = wsum_ref[...]
    at = adj_ref[...]
    deg = 1.0 + jnp.sum(wt, axis=1, keepdims=True)
    dinv = jnp.where(deg > 0, lax.rsqrt(deg), 0.0)
    h1 = jnp.dot(x_ref[...], wgcn_ref[...], preferred_element_type=jnp.float32)
    u = jnp.dot(wt, dinv * h1, preferred_element_type=jnp.float32)
    hg = dinv * u + h1 / deg + bgcn_ref[...]
    cnt = jnp.sum(at, axis=1, keepdims=True)
    sagg = jnp.dot(at, hg, preferred_element_type=jnp.float32)
    mean = sagg / jnp.maximum(cnt, 1.0)
    o = (jnp.dot(mean, wl_ref[...], preferred_element_type=jnp.float32)
         + bl_ref[...]
         + jnp.dot(hg, wr_ref[...], preferred_element_type=jnp.float32))
    nrm = jnp.sqrt(jnp.sum(o * o, axis=1, keepdims=True))
    out_ref[...] = o / jnp.maximum(nrm, 1e-12)


def _tc_dense(wsum, adj, x1k, W_gcn, b_gcn, Wl1, bl1, Wr1):
    return pl.pallas_call(
        _tc_body,
        out_shape=jax.ShapeDtypeStruct((N1, 128), jnp.float32),
    )(wsum, adj, x1k, W_gcn, b_gcn, Wl1, bl1, Wr1)


def kernel(x, attr, W_gcn, b_gcn, Wl0, bl0, Wr0, Wl1, bl1, Wr1,
           edge_index_0, e_id_0, edge_index_1, e_id_1):
    del Wl0, bl0, Wr0, edge_index_0, e_id_0
    ep = NS * EPT
    pad = ep - E1
    row = jnp.concatenate([edge_index_1[0], jnp.zeros((pad,), jnp.int32)])
    col = jnp.concatenate([edge_index_1[1], jnp.full((pad,), N1, jnp.int32)])
    eid = jnp.concatenate([e_id_1, jnp.zeros((pad,), jnp.int32)])
    row = row.reshape(NS, NCHUNK, 128)
    col = col.reshape(NS, NCHUNK, 128)
    eid = eid.reshape(NS, NCHUNK, 128)

    zeros_in = jnp.zeros((SLICE,), jnp.float32)
    ones_in = jnp.ones((NCHUNK, 128), jnp.float32)
    acc = _make_sc_build()(row, col, eid, attr.reshape(-1), zeros_in, ones_in)
    wsum = acc[:N1 * N1].reshape(N1, N1)
    adj = acc[CORE_OUT:CORE_OUT + N1 * N1].reshape(N1, N1)

    return _tc_dense(wsum, adj, x[:N1],
                     W_gcn, b_gcn.reshape(1, 128), Wl1, bl1.reshape(1, 128), Wr1)


# speedup vs baseline: 33.9732x; 1.2061x over previous
import functools

import jax
import jax.numpy as jnp
from jax import lax
from jax.experimental import pallas as pl
from jax.experimental.pallas import tpu as pltpu
from jax.experimental.pallas import tpu_sc as plsc

N1 = 1000
NP = 1024
E1 = 100000
NC, NS, L = 2, 16, 16
EPT = 6272
EPT_LAST = E1 - (NS - 1) * EPT
NCHUNK = EPT // 128
NPH = 2
HROW = N1 // NPH
DUMP = HROW * NP
ACC = DUMP + 256
SLICE = ACC // NS
CORE_OUT = NPH * DUMP + (ACC - DUMP)


def _sc_body(rowi_hbm, coli_hbm, eid_hbm, attr_hbm, zeros_hbm, ones_hbm,
             wsum_out, adj_out,
             row_v, col_v, eid_v, flat_v, val_v, zbuf_v, stage_v, acc_sh,
             sem, gsem):
    c = lax.axis_index("c")
    s = lax.axis_index("s")
    lane = lax.iota(jnp.int32, L)

    base = s * EPT

    @pl.when(s < NS - 1)
    def _full_slices():
        ld = [pltpu.make_async_copy(rowi_hbm.at[pl.ds(base, EPT)], row_v,
                                    sem),
              pltpu.make_async_copy(coli_hbm.at[pl.ds(base, EPT)], col_v,
                                    sem),
              pltpu.make_async_copy(eid_hbm.at[pl.ds(base, EPT)], eid_v,
                                    sem)]
        for d in ld:
            d.start()
        for d in ld:
            d.wait()

    @pl.when(s == NS - 1)
    def _last_slice():
        def zfill(i, carry):
            eid_v[pl.ds(EPT_LAST + i * L, L)] = jnp.zeros((L,), jnp.int32)
            return carry
        lax.fori_loop(0, (EPT - EPT_LAST) // L, zfill, 0)
        ld = [pltpu.make_async_copy(rowi_hbm.at[pl.ds(base, EPT_LAST)],
                                    row_v.at[pl.ds(0, EPT_LAST)], sem),
              pltpu.make_async_copy(coli_hbm.at[pl.ds(base, EPT_LAST)],
                                    col_v.at[pl.ds(0, EPT_LAST)], sem),
              pltpu.make_async_copy(eid_hbm.at[pl.ds(base, EPT_LAST)],
                                    eid_v.at[pl.ds(0, EPT_LAST)], sem)]
        for d in ld:
            d.start()
        for d in ld:
            d.wait()

    ldc = [pltpu.make_async_copy(zeros_hbm, zbuf_v, sem),
           pltpu.make_async_copy(ones_hbm, val_v, sem)]
    for d in ldc:
        d.start()
    for d in ldc:
        d.wait()

    @pl.when(c == 0)
    def _gather_w():
        with jax.named_scope("sc_gather_w"):
            gs = [pltpu.make_async_copy(
                      attr_hbm.at[eid_v.at[pl.ds(j * 128, 128)]],
                      val_v.at[j], gsem)
                  for j in range(NCHUNK)]
            for d in gs:
                d.start()
            for d in gs:
                d.wait()

    bound = jnp.where(s == NS - 1, EPT_LAST, EPT)
    outb = []
    for p in range(NPH):
        with jax.named_scope(f"sc_zero_{p}"):
            pltpu.sync_copy(zbuf_v, acc_sh.at[pl.ds(s * SLICE, SLICE)])

        with jax.named_scope(f"sc_flat_{p}"):
            def fbody(j, carry):
                def kbody(k, carry2):
                    off = k * L
                    r = col_v[pl.ds(j * 128 + off, L)]
                    cc = row_v[pl.ds(j * 128 + off, L)]
                    rr = r - p * HROW
                    flat = rr * NP + cc
                    ok = ((rr >= 0) & (rr < HROW) & (cc >= 0) & (cc < N1)
                          & (j * 128 + off + lane < bound))
                    flat_v[j, pl.ds(off, L)] = jnp.where(
                        ok, flat, DUMP + off + lane)
                    return carry2
                return lax.fori_loop(0, 128 // L, kbody, carry)
            lax.fori_loop(0, NCHUNK, fbody, 0)
            plsc.subcore_barrier()

        with jax.named_scope(f"sc_scatter_{p}"):
            sc = [pltpu.make_async_copy(val_v.at[j],
                                        acc_sh.at[flat_v.at[j]], sem)
                  for j in range(NCHUNK)]
            for d in sc:
                d.start(add=True)
            for d in sc:
                d.wait()
            for d in outb:
                d.wait()
            outb = []
            plsc.subcore_barrier()

        with jax.named_scope(f"sc_out_{p}"):
            pltpu.sync_copy(acc_sh.at[pl.ds(s * SLICE, SLICE)], stage_v)
            dst = pl.ds(p * DUMP + s * SLICE, SLICE)

            @pl.when(c == 0)
            def _out_w():
                d = pltpu.make_async_copy(stage_v, wsum_out.at[dst], gsem)
                d.start()

            @pl.when(c == 1)
            def _out_a():
                d = pltpu.make_async_copy(stage_v, adj_out.at[dst], gsem)
                d.start()

            outb = [pltpu.make_async_copy(stage_v, wsum_out.at[dst], gsem)]
    for d in outb:
        d.wait()


def _make_sc_build():
    return functools.partial(
        pl.kernel,
        out_type=[jax.ShapeDtypeStruct((CORE_OUT,), jnp.float32),
                  jax.ShapeDtypeStruct((CORE_OUT,), jnp.float32)],
        mesh=plsc.VectorSubcoreMesh(core_axis_name="c", subcore_axis_name="s",
                                    num_cores=NC, num_subcores=NS),
        scratch_types=[
            pltpu.VMEM((EPT,), jnp.int32),
            pltpu.VMEM((EPT,), jnp.int32),
            pltpu.VMEM((EPT,), jnp.int32),
            pltpu.VMEM((NCHUNK, 128), jnp.int32),
            pltpu.VMEM((NCHUNK, 128), jnp.float32),
            pltpu.VMEM((SLICE,), jnp.float32),
            pltpu.VMEM((SLICE,), jnp.float32),
            pltpu.VMEM_SHARED((ACC,), jnp.float32),
            pltpu.SemaphoreType.DMA,
            pltpu.SemaphoreType.DMA,
        ],
    )(_sc_body)


def _tc_body(wsum_ref, adj_ref, x_ref, wgcn_ref, bgcn_ref,
             wl_ref, bl_ref, wr_ref, out_ref):
    wt = wsum_ref[...]
    at = adj_ref[...]
    deg = 1.0 + jnp.sum(wt, axis=1, keepdims=True)
    dinv = lax.rsqrt(deg)
    h1 = jnp.dot(x_ref[...], wgcn_ref[...], preferred_element_type=jnp.float32)
    dinv_p = jnp.concatenate([dinv, jnp.zeros((NP - N1, 1), jnp.float32)],
                             axis=0)
    u = jnp.dot(wt, dinv_p * h1, preferred_element_type=jnp.float32)
    hg = dinv * u + h1[:N1] / deg + bgcn_ref[...]
    cnt = jnp.sum(at, axis=1, keepdims=True)
    hg_p = jnp.concatenate([hg, jnp.zeros((NP - N1, 128), jnp.float32)],
                           axis=0)
    sagg = jnp.dot(at, hg_p, preferred_element_type=jnp.float32)
    mean = sagg / jnp.maximum(cnt, 1.0)
    o = (jnp.dot(mean, wl_ref[...], preferred_element_type=jnp.float32)
         + bl_ref[...]
         + jnp.dot(hg, wr_ref[...], preferred_element_type=jnp.float32))
    nrm = jnp.sqrt(jnp.sum(o * o, axis=1, keepdims=True))
    out_ref[...] = o / jnp.maximum(nrm, 1e-12)


def _tc_dense(wsum, adj, xp, W_gcn, b_gcn, Wl1, bl1, Wr1):
    return pl.pallas_call(
        _tc_body,
        out_shape=jax.ShapeDtypeStruct((N1, 128), jnp.float32),
    )(wsum, adj, xp, W_gcn, b_gcn, Wl1, bl1, Wr1)


def kernel(x, attr, W_gcn, b_gcn, Wl0, bl0, Wr0, Wl1, bl1, Wr1,
           edge_index_0, e_id_0, edge_index_1, e_id_1):
    del Wl0, bl0, Wr0, edge_index_0, e_id_0
    zeros_in = jnp.zeros((SLICE,), jnp.float32)
    ones_in = jnp.ones((NCHUNK, 128), jnp.float32)

    wsum_f, adj_f = _make_sc_build()(
        edge_index_1[0], edge_index_1[1], e_id_1, attr.reshape(-1),
        zeros_in, ones_in)
    wsum = wsum_f[:N1 * NP].reshape(N1, NP)
    adj = adj_f[:N1 * NP].reshape(N1, NP)

    return _tc_dense(wsum, adj, x[:NP],
                     W_gcn, b_gcn.reshape(1, 128), Wl1, bl1.reshape(1, 128),
                     Wr1)
